# baseline (speedup 1.0000x reference)
import functools

import jax
import jax.numpy as jnp
from jax import lax
from jax.experimental import pallas as pl
from jax.experimental.pallas import tpu as pltpu
from jax.experimental.pallas import tpu_sc as plsc

B = 512
D = 128
L = 16
NC = 2
NS = 16
NW = NC * NS
RW = B // NW
CH = B // L
KPOS = 3
KNEG = 5
MARGIN = 1.0
BOUNDARY = 5
EPS_POS = 1e-07

_MESH = plsc.VectorSubcoreMesh(core_axis_name="c", subcore_axis_name="s",
                               num_cores=NC, num_subcores=NS)


def _iota16():
    return lax.broadcasted_iota(jnp.int32, (L,), 0)


def _splat_max(v):
    m = plsc.cummax(v)
    m = lax.rev(m, (0,))
    return plsc.cummax(m)


def _splat_min(v):
    return -_splat_max(-v)


def _splat_at(v, selj, lowest):
    return _splat_max(jnp.where(selj, v, lowest))


def _popcount(mask):
    return plsc.all_reduce_population_count(mask)


def _dist_kernel(x_ref, d_ref, dt_ref):
    x = x_ref[...]
    dot = lax.dot_general(x, x, (((1,), (1,)), ((), ())),
                          preferred_element_type=jnp.float32)
    rows = lax.broadcasted_iota(jnp.int32, (B, B), 0)
    cols = lax.broadcasted_iota(jnp.int32, (B, B), 1)
    eye = rows == cols
    diag_m = jnp.where(eye, dot, 0.0)
    sq_col = jnp.sum(diag_m, axis=1, keepdims=True)
    sq_row = jnp.sum(diag_m, axis=0, keepdims=True)
    d = sq_row - 2.0 * dot + sq_col
    d = jnp.maximum(d, 0.0)
    d_ref[...] = jnp.where(d == 0.0, 0.0, jnp.sqrt(d))
    dt = sq_col - 2.0 * dot + sq_row
    dt = jnp.maximum(dt, 0.0)
    dt_ref[...] = jnp.where(dt == 0.0, 0.0, jnp.sqrt(dt))


def _row_kernel(dist_hbm, tgt_hbm, p0_hbm, p1_hbm, p2_hbm, anch_hbm,
                rowblk, tgt_v, mrow, stage):
    wid = lax.axis_index("s") * NC + lax.axis_index("c")
    base = wid * RW
    pltpu.sync_copy(tgt_hbm, tgt_v)
    pltpu.sync_copy(dist_hbm.at[pl.ds(base, RW)], rowblk)

    neg_inf = jnp.float32(-jnp.inf)

    def per_row(j, carry):
        p0v, p1v, p2v, anchv = carry
        jv = jnp.full((L,), j, jnp.int32)
        selj = _iota16() == jv
        tmine = tgt_v[pl.ds(base, L)]
        tj = _splat_at(tmine, selj, -1)
        gj = jv + base

        def mask_count(c, cntv):
            dv = rowblk[j, pl.ds(c * L, L)]
            tc = tgt_v[pl.ds(c * L, L)]
            same = tc == tj
            gidx = _iota16() + c * L
            keep = same & (gidx != gj)
            mrow[pl.ds(c * L, L)] = jnp.where(keep, dv, neg_inf)
            return cntv + _popcount(same)

        cntv = lax.fori_loop(0, CH, mask_count,
                             jnp.zeros((L,), jnp.int32))
        anch = cntv < BOUNDARY

        pks = []
        for k in range(KPOS):
            def runmax(c, rm):
                return jnp.maximum(rm, mrow[pl.ds(c * L, L)])
            rm = lax.fori_loop(0, CH, runmax, jnp.full((L,), neg_inf))
            mk = _splat_max(rm)
            pks.append(mk)
            if k < KPOS - 1:
                def excl(c, done, mk=mk):
                    ch = mrow[pl.ds(c * L, L)]
                    eq = ch == mk
                    anyeq = _popcount(eq) > 0
                    lanepos = jnp.where(eq, _iota16(), L)
                    fl = _splat_min(lanepos)
                    take = (done == 0) & anyeq
                    mrow[pl.ds(c * L, L)] = jnp.where(
                        take & (_iota16() == fl), neg_inf, ch)
                    return jnp.where(take, 1, done)
                lax.fori_loop(0, CH, excl, jnp.zeros((L,), jnp.int32))

        p0v = jnp.where(selj, pks[0], p0v)
        p1v = jnp.where(selj, pks[1], p1v)
        p2v = jnp.where(selj, pks[2], p2v)
        anchv = jnp.where(selj, jnp.where(anch, 1.0, 0.0), anchv)
        return (p0v, p1v, p2v, anchv)

    z = jnp.zeros((L,), jnp.float32)
    p0v, p1v, p2v, anchv = lax.fori_loop(0, RW, per_row, (z, z, z, z))

    for vec, out in ((p0v, p0_hbm), (p1v, p1_hbm), (p2v, p2_hbm),
                     (anchv, anch_hbm)):
        stage[...] = vec
        pltpu.sync_copy(stage, out.at[pl.ds(base, RW)])


def _col_kernel(distt_hbm, tgt_hbm, p0_hbm, p1_hbm, p2_hbm, anch_hbm,
                ps_hbm, pn_hbm,
                colblk, tgt_v, p0_v, p1_v, p2_v, anch_v, mcol, stage):
    wid = lax.axis_index("s") * NC + lax.axis_index("c")
    base = wid * RW
    pltpu.sync_copy(tgt_hbm, tgt_v)
    pltpu.sync_copy(distt_hbm.at[pl.ds(base, RW)], colblk)
    pltpu.sync_copy(p0_hbm, p0_v)
    pltpu.sync_copy(p1_hbm, p1_v)
    pltpu.sync_copy(p2_hbm, p2_v)
    pltpu.sync_copy(anch_hbm, anch_v)

    inf = jnp.float32(jnp.inf)
    neg_inf = jnp.float32(-jnp.inf)

    def per_col(j, carry):
        sacc, nacc = carry
        jv = jnp.full((L,), j, jnp.int32)
        tmine = tgt_v[pl.ds(base, L)]
        tn = _splat_at(tmine, _iota16() == jv, -1)

        def mask_col(c, _):
            dv = colblk[j, pl.ds(c * L, L)]
            tc = tgt_v[pl.ds(c * L, L)]
            mcol[pl.ds(c * L, L)] = jnp.where(tc == tn, inf, dv)
            return 0

        lax.fori_loop(0, CH, mask_col, 0)

        th = jnp.full((L,), neg_inf)
        cnt = jnp.zeros((L,), jnp.int32)
        for _ in range(KNEG):
            def runmin(c, rm, th=th):
                ch = mcol[pl.ds(c * L, L)]
                return jnp.minimum(rm, jnp.where(ch > th, ch, inf))
            rm = lax.fori_loop(0, CH, runmin, jnp.full((L,), inf))
            m = _splat_min(rm)

            def runcnt(c, rc, m=m):
                ch = mcol[pl.ds(c * L, L)]
                return rc + _popcount(ch <= m)
            c2 = lax.fori_loop(0, CH, runcnt, jnp.zeros((L,), jnp.int32))
            take = cnt < KNEG
            th = jnp.where(take, m, th)
            cnt = jnp.where(take, c2, cnt)

        def accum(c, sn):
            s, n = sn
            dv = colblk[j, pl.ds(c * L, L)]
            tc = tgt_v[pl.ds(c * L, L)]
            same = tc == tn
            m2 = (~same) & (dv <= th) & (anch_v[pl.ds(c * L, L)] > 0.5)
            for pk_ref in (p0_v, p1_v, p2_v):
                pk = pk_ref[pl.ds(c * L, L)]
                tr = jnp.maximum(pk + MARGIN - dv, 0.0)
                s = s + jnp.where(m2, tr, 0.0)
                n = n + jnp.where(m2 & (tr > EPS_POS), 1.0, 0.0)
            return (s, n)

        return lax.fori_loop(0, CH, accum, (sacc, nacc))

    z = jnp.zeros((L,), jnp.float32)
    sacc, nacc = lax.fori_loop(0, RW, per_col, (z, z))

    stage[...] = sacc
    pltpu.sync_copy(stage, ps_hbm.at[wid])
    stage[...] = nacc
    pltpu.sync_copy(stage, pn_hbm.at[wid])


def _final_kernel(ps_ref, pn_ref, out_ref):
    s = jnp.sum(ps_ref[...])
    n = jnp.sum(pn_ref[...])
    out_ref[...] = (s / (n + EPS_POS)).reshape(1, 1)


_row_phase = functools.partial(
    pl.kernel,
    out_type=(
        jax.ShapeDtypeStruct((B,), jnp.float32),
        jax.ShapeDtypeStruct((B,), jnp.float32),
        jax.ShapeDtypeStruct((B,), jnp.float32),
        jax.ShapeDtypeStruct((B,), jnp.float32),
    ),
    mesh=_MESH,
    compiler_params=pltpu.CompilerParams(needs_layout_passes=False),
    scratch_types=[
        pltpu.VMEM((RW, B), jnp.float32),
        pltpu.VMEM((B,), jnp.int32),
        pltpu.VMEM((B,), jnp.float32),
        pltpu.VMEM((L,), jnp.float32),
    ],
)(_row_kernel)

_col_phase = functools.partial(
    pl.kernel,
    out_type=(
        jax.ShapeDtypeStruct((NW, L), jnp.float32),
        jax.ShapeDtypeStruct((NW, L), jnp.float32),
    ),
    mesh=_MESH,
    compiler_params=pltpu.CompilerParams(needs_layout_passes=False),
    scratch_types=[
        pltpu.VMEM((RW, B), jnp.float32),
        pltpu.VMEM((B,), jnp.int32),
        pltpu.VMEM((B,), jnp.float32),
        pltpu.VMEM((B,), jnp.float32),
        pltpu.VMEM((B,), jnp.float32),
        pltpu.VMEM((B,), jnp.float32),
        pltpu.VMEM((B,), jnp.float32),
        pltpu.VMEM((L,), jnp.float32),
    ],
)(_col_kernel)


def kernel(input, target):
    dist, distt = pl.pallas_call(
        _dist_kernel,
        out_shape=(jax.ShapeDtypeStruct((B, B), jnp.float32),
                   jax.ShapeDtypeStruct((B, B), jnp.float32)),
    )(input)
    p0, p1, p2, anch = _row_phase(dist, target)
    ps, pn = _col_phase(distt, target, p0, p1, p2, anch)
    out = pl.pallas_call(
        _final_kernel,
        out_shape=jax.ShapeDtypeStruct((1, 1), jnp.float32),
    )(ps, pn)
    return out.reshape(())


# speedup vs baseline: 1.4639x; 1.4639x over previous
import functools

import jax
import jax.numpy as jnp
from jax import lax
from jax.experimental import pallas as pl
from jax.experimental.pallas import tpu as pltpu
from jax.experimental.pallas import tpu_sc as plsc

B = 512
D = 128
L = 16
NC = 2
NS = 16
NW = NC * NS
RW = B // NW
CH = B // L
KPOS = 3
KNEG = 5
MARGIN = 1.0
BOUNDARY = 5
EPS_POS = 1e-07

_MESH = plsc.VectorSubcoreMesh(core_axis_name="c", subcore_axis_name="s",
                               num_cores=NC, num_subcores=NS)


def _iota16():
    return lax.broadcasted_iota(jnp.int32, (L,), 0)


def _splat_max(v):
    m = plsc.cummax(v)
    m = lax.rev(m, (0,))
    return plsc.cummax(m)


def _splat_min(v):
    return -_splat_max(-v)


def _splat_at(v, selj, lowest):
    return _splat_max(jnp.where(selj, v, lowest))


def _popcount(mask):
    return plsc.all_reduce_population_count(mask)


def _dist_kernel(x_ref, d_ref, dt_ref):
    x = x_ref[...]
    dot = lax.dot_general(x, x, (((1,), (1,)), ((), ())),
                          preferred_element_type=jnp.float32)
    rows = lax.broadcasted_iota(jnp.int32, (B, B), 0)
    cols = lax.broadcasted_iota(jnp.int32, (B, B), 1)
    eye = rows == cols
    diag_m = jnp.where(eye, dot, 0.0)
    sq_col = jnp.sum(diag_m, axis=1, keepdims=True)
    sq_row = jnp.sum(diag_m, axis=0, keepdims=True)
    d = sq_row - 2.0 * dot + sq_col
    d = jnp.maximum(d, 0.0)
    d_ref[...] = jnp.where(d == 0.0, 0.0, jnp.sqrt(d))
    dt = sq_col - 2.0 * dot + sq_row
    dt = jnp.maximum(dt, 0.0)
    dt_ref[...] = jnp.where(dt == 0.0, 0.0, jnp.sqrt(dt))


def _row_kernel(dist_hbm, tgt_hbm, p0_hbm, p1_hbm, p2_hbm, anch_hbm,
                rowblk, tgt_v, mrow, stage):
    wid = lax.axis_index("s") * NC + lax.axis_index("c")
    base = wid * RW
    pltpu.sync_copy(tgt_hbm, tgt_v)
    pltpu.sync_copy(dist_hbm.at[pl.ds(base, RW)], rowblk)

    neg_inf = jnp.float32(-jnp.inf)

    def per_row(j, carry):
        p0v, p1v, p2v, anchv = carry
        jv = jnp.full((L,), j, jnp.int32)
        selj = _iota16() == jv
        tmine = tgt_v[pl.ds(base, L)]
        tj = _splat_at(tmine, selj, -1)
        gj = jv + base

        cntv = jnp.zeros((L,), jnp.int32)
        for c in range(CH):
            dv = rowblk[j, pl.ds(c * L, L)]
            tc = tgt_v[pl.ds(c * L, L)]
            same = tc == tj
            gidx = _iota16() + c * L
            keep = same & (gidx != gj)
            mrow[pl.ds(c * L, L)] = jnp.where(keep, dv, neg_inf)
            cntv = cntv + _popcount(same)
        anch = cntv < BOUNDARY

        pks = []
        for k in range(KPOS):
            rm = jnp.full((L,), neg_inf)
            for c in range(CH):
                rm = jnp.maximum(rm, mrow[pl.ds(c * L, L)])
            mk = _splat_max(rm)
            pks.append(mk)
            if k < KPOS - 1:
                done = jnp.zeros((L,), jnp.int32)
                for c in range(CH):
                    ch = mrow[pl.ds(c * L, L)]
                    eq = ch == mk
                    anyeq = _popcount(eq) > 0
                    lanepos = jnp.where(eq, _iota16(), L)
                    fl = _splat_min(lanepos)
                    take = (done == 0) & anyeq
                    mrow[pl.ds(c * L, L)] = jnp.where(
                        take & (_iota16() == fl), neg_inf, ch)
                    done = jnp.where(take, 1, done)

        p0v = jnp.where(selj, pks[0], p0v)
        p1v = jnp.where(selj, pks[1], p1v)
        p2v = jnp.where(selj, pks[2], p2v)
        anchv = jnp.where(selj, jnp.where(anch, 1.0, 0.0), anchv)
        return (p0v, p1v, p2v, anchv)

    z = jnp.zeros((L,), jnp.float32)
    p0v, p1v, p2v, anchv = lax.fori_loop(0, RW, per_row, (z, z, z, z))

    for vec, out in ((p0v, p0_hbm), (p1v, p1_hbm), (p2v, p2_hbm),
                     (anchv, anch_hbm)):
        stage[...] = vec
        pltpu.sync_copy(stage, out.at[pl.ds(base, RW)])


def _col_kernel(distt_hbm, tgt_hbm, p0_hbm, p1_hbm, p2_hbm, anch_hbm,
                ps_hbm, pn_hbm,
                colblk, tgt_v, p0_v, p1_v, p2_v, anch_v, mcol, stage):
    wid = lax.axis_index("s") * NC + lax.axis_index("c")
    base = wid * RW
    pltpu.sync_copy(tgt_hbm, tgt_v)
    pltpu.sync_copy(distt_hbm.at[pl.ds(base, RW)], colblk)
    pltpu.sync_copy(p0_hbm, p0_v)
    pltpu.sync_copy(p1_hbm, p1_v)
    pltpu.sync_copy(p2_hbm, p2_v)
    pltpu.sync_copy(anch_hbm, anch_v)

    inf = jnp.float32(jnp.inf)
    neg_inf = jnp.float32(-jnp.inf)

    def per_col(j, carry):
        sacc, nacc = carry
        jv = jnp.full((L,), j, jnp.int32)
        tmine = tgt_v[pl.ds(base, L)]
        tn = _splat_at(tmine, _iota16() == jv, -1)

        for c in range(CH):
            dv = colblk[j, pl.ds(c * L, L)]
            tc = tgt_v[pl.ds(c * L, L)]
            mcol[pl.ds(c * L, L)] = jnp.where(tc == tn, inf, dv)

        th = jnp.full((L,), neg_inf)
        cnt = jnp.zeros((L,), jnp.int32)
        for _ in range(KNEG):
            rm = jnp.full((L,), inf)
            for c in range(CH):
                ch = mcol[pl.ds(c * L, L)]
                rm = jnp.minimum(rm, jnp.where(ch > th, ch, inf))
            m = _splat_min(rm)
            c2 = jnp.zeros((L,), jnp.int32)
            for c in range(CH):
                ch = mcol[pl.ds(c * L, L)]
                c2 = c2 + _popcount(ch <= m)
            take = cnt < KNEG
            th = jnp.where(take, m, th)
            cnt = jnp.where(take, c2, cnt)

        s, n = sacc, nacc
        for c in range(CH):
            dv = colblk[j, pl.ds(c * L, L)]
            tc = tgt_v[pl.ds(c * L, L)]
            same = tc == tn
            m2 = (~same) & (dv <= th) & (anch_v[pl.ds(c * L, L)] > 0.5)
            for pk_ref in (p0_v, p1_v, p2_v):
                pk = pk_ref[pl.ds(c * L, L)]
                tr = jnp.maximum(pk + MARGIN - dv, 0.0)
                s = s + jnp.where(m2, tr, 0.0)
                n = n + jnp.where(m2 & (tr > EPS_POS), 1.0, 0.0)
        return (s, n)

    z = jnp.zeros((L,), jnp.float32)
    sacc, nacc = lax.fori_loop(0, RW, per_col, (z, z))

    stage[...] = sacc
    pltpu.sync_copy(stage, ps_hbm.at[wid])
    stage[...] = nacc
    pltpu.sync_copy(stage, pn_hbm.at[wid])


def _final_kernel(ps_ref, pn_ref, out_ref):
    s = jnp.sum(ps_ref[...])
    n = jnp.sum(pn_ref[...])
    out_ref[...] = (s / (n + EPS_POS)).reshape(1, 1)


_row_phase = functools.partial(
    pl.kernel,
    out_type=(
        jax.ShapeDtypeStruct((B,), jnp.float32),
        jax.ShapeDtypeStruct((B,), jnp.float32),
        jax.ShapeDtypeStruct((B,), jnp.float32),
        jax.ShapeDtypeStruct((B,), jnp.float32),
    ),
    mesh=_MESH,
    compiler_params=pltpu.CompilerParams(needs_layout_passes=False),
    scratch_types=[
        pltpu.VMEM((RW, B), jnp.float32),
        pltpu.VMEM((B,), jnp.int32),
        pltpu.VMEM((B,), jnp.float32),
        pltpu.VMEM((L,), jnp.float32),
    ],
)(_row_kernel)

_col_phase = functools.partial(
    pl.kernel,
    out_type=(
        jax.ShapeDtypeStruct((NW, L), jnp.float32),
        jax.ShapeDtypeStruct((NW, L), jnp.float32),
    ),
    mesh=_MESH,
    compiler_params=pltpu.CompilerParams(needs_layout_passes=False),
    scratch_types=[
        pltpu.VMEM((RW, B), jnp.float32),
        pltpu.VMEM((B,), jnp.int32),
        pltpu.VMEM((B,), jnp.float32),
        pltpu.VMEM((B,), jnp.float32),
        pltpu.VMEM((B,), jnp.float32),
        pltpu.VMEM((B,), jnp.float32),
        pltpu.VMEM((B,), jnp.float32),
        pltpu.VMEM((L,), jnp.float32),
    ],
)(_col_kernel)


def kernel(input, target):
    dist, distt = pl.pallas_call(
        _dist_kernel,
        out_shape=(jax.ShapeDtypeStruct((B, B), jnp.float32),
                   jax.ShapeDtypeStruct((B, B), jnp.float32)),
    )(input)
    p0, p1, p2, anch = _row_phase(dist, target)
    ps, pn = _col_phase(distt, target, p0, p1, p2, anch)
    out = pl.pallas_call(
        _final_kernel,
        out_shape=jax.ShapeDtypeStruct((1, 1), jnp.float32),
    )(ps, pn)
    return out.reshape(())


# speedup vs baseline: 1.7600x; 1.2023x over previous
import functools

import jax
import jax.numpy as jnp
from jax import lax
from jax.experimental import pallas as pl
from jax.experimental.pallas import tpu as pltpu
from jax.experimental.pallas import tpu_sc as plsc

B = 512
D = 128
L = 16
NC = 2
NS = 16
NW = NC * NS
RW = B // NW
CH = B // L
KPOS = 3
KNEG = 5
MARGIN = 1.0
BOUNDARY = 5
EPS_POS = 1e-07

_MESH = plsc.VectorSubcoreMesh(core_axis_name="c", subcore_axis_name="s",
                               num_cores=NC, num_subcores=NS)


def _iota16():
    return lax.broadcasted_iota(jnp.int32, (L,), 0)


def _splat_max(v):
    m = plsc.cummax(v)
    m = lax.rev(m, (0,))
    return plsc.cummax(m)


def _splat_min(v):
    return -_splat_max(-v)


def _splat_at(v, selj, lowest):
    return _splat_max(jnp.where(selj, v, lowest))


def _popcount(mask):
    return plsc.all_reduce_population_count(mask)


def _dist_kernel(x_ref, d_ref, dt_ref):
    x = x_ref[...]
    dot = lax.dot_general(x, x, (((1,), (1,)), ((), ())),
                          preferred_element_type=jnp.float32)
    rows = lax.broadcasted_iota(jnp.int32, (B, B), 0)
    cols = lax.broadcasted_iota(jnp.int32, (B, B), 1)
    eye = rows == cols
    diag_m = jnp.where(eye, dot, 0.0)
    sq_col = jnp.sum(diag_m, axis=1, keepdims=True)
    sq_row = jnp.sum(diag_m, axis=0, keepdims=True)
    d = sq_row - 2.0 * dot + sq_col
    d = jnp.maximum(d, 0.0)
    d_ref[...] = jnp.where(d == 0.0, 0.0, jnp.sqrt(d))
    dt = sq_col - 2.0 * dot + sq_row
    dt = jnp.maximum(dt, 0.0)
    dt_ref[...] = jnp.where(dt == 0.0, 0.0, jnp.sqrt(dt))


def _row_kernel(dist_hbm, tgt_hbm, p0_hbm, p1_hbm, p2_hbm, anch_hbm,
                rowblk, tgt_v, stage):
    wid = lax.axis_index("s") * NC + lax.axis_index("c")
    base = wid * RW
    pltpu.sync_copy(tgt_hbm, tgt_v)
    pltpu.sync_copy(dist_hbm.at[pl.ds(base, RW)], rowblk)

    neg_inf = jnp.float32(-jnp.inf)

    def per_row(j, carry):
        p0v, p1v, p2v, anchv = carry
        jv = jnp.full((L,), j, jnp.int32)
        selj = _iota16() == jv
        tmine = tgt_v[pl.ds(base, L)]
        tj = _splat_at(tmine, selj, -1)
        gj = jv + base

        cntv = jnp.zeros((L,), jnp.int32)
        m1 = jnp.full((L,), neg_inf)
        m2 = jnp.full((L,), neg_inf)
        m3 = jnp.full((L,), neg_inf)
        for c in range(CH):
            dv = rowblk[j, pl.ds(c * L, L)]
            tc = tgt_v[pl.ds(c * L, L)]
            same = tc == tj
            gidx = _iota16() + c * L
            keep = same & (gidx != gj)
            new = jnp.where(keep, dv, neg_inf)
            cntv = cntv + _popcount(same)
            hi = jnp.maximum(m1, new); new = jnp.minimum(m1, new); m1 = hi
            hi = jnp.maximum(m2, new); new = jnp.minimum(m2, new); m2 = hi
            m3 = jnp.maximum(m3, new)
        anch = cntv < BOUNDARY

        r, _ = plsc.sort_key_val(m1, m1, descending=True)
        for mi in (m2, m3):
            s, _ = plsc.sort_key_val(mi, mi, descending=True)
            r = jnp.maximum(r, lax.rev(s, (0,)))
            r, _ = plsc.sort_key_val(r, r, descending=True)
        pks = [_splat_at(r, _iota16() == k, neg_inf) for k in range(KPOS)]

        p0v = jnp.where(selj, pks[0], p0v)
        p1v = jnp.where(selj, pks[1], p1v)
        p2v = jnp.where(selj, pks[2], p2v)
        anchv = jnp.where(selj, jnp.where(anch, 1.0, 0.0), anchv)
        return (p0v, p1v, p2v, anchv)

    z = jnp.zeros((L,), jnp.float32)
    p0v, p1v, p2v, anchv = lax.fori_loop(0, RW, per_row, (z, z, z, z))

    for vec, out in ((p0v, p0_hbm), (p1v, p1_hbm), (p2v, p2_hbm),
                     (anchv, anch_hbm)):
        stage[...] = vec
        pltpu.sync_copy(stage, out.at[pl.ds(base, RW)])


def _col_kernel(distt_hbm, tgt_hbm, p0_hbm, p1_hbm, p2_hbm, anch_hbm,
                ps_hbm, pn_hbm,
                colblk, tgt_v, p0_v, p1_v, p2_v, anch_v, stage):
    wid = lax.axis_index("s") * NC + lax.axis_index("c")
    base = wid * RW
    pltpu.sync_copy(tgt_hbm, tgt_v)
    pltpu.sync_copy(distt_hbm.at[pl.ds(base, RW)], colblk)
    pltpu.sync_copy(p0_hbm, p0_v)
    pltpu.sync_copy(p1_hbm, p1_v)
    pltpu.sync_copy(p2_hbm, p2_v)
    pltpu.sync_copy(anch_hbm, anch_v)

    inf = jnp.float32(jnp.inf)
    neg_inf = jnp.float32(-jnp.inf)

    def per_col(j, carry):
        sacc, nacc = carry
        jv = jnp.full((L,), j, jnp.int32)
        tmine = tgt_v[pl.ds(base, L)]
        tn = _splat_at(tmine, _iota16() == jv, -1)

        a1 = jnp.full((L,), inf)
        a2 = jnp.full((L,), inf)
        a3 = jnp.full((L,), inf)
        a4 = jnp.full((L,), inf)
        a5 = jnp.full((L,), inf)
        for c in range(CH):
            dv = colblk[j, pl.ds(c * L, L)]
            tc = tgt_v[pl.ds(c * L, L)]
            new = jnp.where(tc == tn, inf, dv)
            lo = jnp.minimum(a1, new); new = jnp.maximum(a1, new); a1 = lo
            lo = jnp.minimum(a2, new); new = jnp.maximum(a2, new); a2 = lo
            lo = jnp.minimum(a3, new); new = jnp.maximum(a3, new); a3 = lo
            lo = jnp.minimum(a4, new); new = jnp.maximum(a4, new); a4 = lo
            a5 = jnp.minimum(a5, new)
        r, _ = plsc.sort_key_val(a1, a1)
        for ai in (a2, a3, a4, a5):
            s, _ = plsc.sort_key_val(ai, ai)
            r = jnp.minimum(r, lax.rev(s, (0,)))
            r, _ = plsc.sort_key_val(r, r)
        th = _splat_at(r, _iota16() == (KNEG - 1), neg_inf)

        s, n = sacc, nacc
        for c in range(CH):
            dv = colblk[j, pl.ds(c * L, L)]
            tc = tgt_v[pl.ds(c * L, L)]
            same = tc == tn
            m2 = (~same) & (dv <= th) & (anch_v[pl.ds(c * L, L)] > 0.5)
            for pk_ref in (p0_v, p1_v, p2_v):
                pk = pk_ref[pl.ds(c * L, L)]
                tr = jnp.maximum(pk + MARGIN - dv, 0.0)
                s = s + jnp.where(m2, tr, 0.0)
                n = n + jnp.where(m2 & (tr > EPS_POS), 1.0, 0.0)
        return (s, n)

    z = jnp.zeros((L,), jnp.float32)
    sacc, nacc = lax.fori_loop(0, RW, per_col, (z, z))

    stage[...] = sacc
    pltpu.sync_copy(stage, ps_hbm.at[wid])
    stage[...] = nacc
    pltpu.sync_copy(stage, pn_hbm.at[wid])


def _final_kernel(ps_ref, pn_ref, out_ref):
    s = jnp.sum(ps_ref[...])
    n = jnp.sum(pn_ref[...])
    out_ref[...] = (s / (n + EPS_POS)).reshape(1, 1)


_row_phase = functools.partial(
    pl.kernel,
    out_type=(
        jax.ShapeDtypeStruct((B,), jnp.float32),
        jax.ShapeDtypeStruct((B,), jnp.float32),
        jax.ShapeDtypeStruct((B,), jnp.float32),
        jax.ShapeDtypeStruct((B,), jnp.float32),
    ),
    mesh=_MESH,
    compiler_params=pltpu.CompilerParams(needs_layout_passes=False),
    scratch_types=[
        pltpu.VMEM((RW, B), jnp.float32),
        pltpu.VMEM((B,), jnp.int32),
        pltpu.VMEM((L,), jnp.float32),
    ],
)(_row_kernel)

_col_phase = functools.partial(
    pl.kernel,
    out_type=(
        jax.ShapeDtypeStruct((NW, L), jnp.float32),
        jax.ShapeDtypeStruct((NW, L), jnp.float32),
    ),
    mesh=_MESH,
    compiler_params=pltpu.CompilerParams(needs_layout_passes=False),
    scratch_types=[
        pltpu.VMEM((RW, B), jnp.float32),
        pltpu.VMEM((B,), jnp.int32),
        pltpu.VMEM((B,), jnp.float32),
        pltpu.VMEM((B,), jnp.float32),
        pltpu.VMEM((B,), jnp.float32),
        pltpu.VMEM((B,), jnp.float32),
        pltpu.VMEM((L,), jnp.float32),
    ],
)(_col_kernel)


def kernel(input, target):
    dist, distt = pl.pallas_call(
        _dist_kernel,
        out_shape=(jax.ShapeDtypeStruct((B, B), jnp.float32),
                   jax.ShapeDtypeStruct((B, B), jnp.float32)),
    )(input)
    p0, p1, p2, anch = _row_phase(dist, target)
    ps, pn = _col_phase(distt, target, p0, p1, p2, anch)
    out = pl.pallas_call(
        _final_kernel,
        out_shape=jax.ShapeDtypeStruct((1, 1), jnp.float32),
    )(ps, pn)
    return out.reshape(())


# speedup vs baseline: 2.5770x; 1.4642x over previous
import functools

import jax
import jax.numpy as jnp
from jax import lax
from jax.experimental import pallas as pl
from jax.experimental.pallas import tpu as pltpu
from jax.experimental.pallas import tpu_sc as plsc

B = 512
D = 128
L = 16
NC = 2
NS = 16
NW = NC * NS
RW = B // NW
CH = B // L
PROWS = 8
KPOS = 3
KNEG = 5
MARGIN = 1.0
BOUNDARY = 5.0
EPS_POS = 1e-07

_MESH = plsc.VectorSubcoreMesh(core_axis_name="c", subcore_axis_name="s",
                               num_cores=NC, num_subcores=NS)


def _iota16():
    return lax.broadcasted_iota(jnp.int32, (L,), 0)


def _splat_max(v):
    m = plsc.cummax(v)
    m = lax.rev(m, (0,))
    return plsc.cummax(m)


def _splat_at(v, selj, lowest):
    return _splat_max(jnp.where(selj, v, lowest))


def _prep_kernel(x_ref, tcol_ref, trow_ref, dtm_ref, pos_ref):
    x = x_ref[...]
    tcol = tcol_ref[...]
    trow = trow_ref[...]
    dot = lax.dot_general(x, x, (((1,), (1,)), ((), ())),
                          preferred_element_type=jnp.float32)
    rows = lax.broadcasted_iota(jnp.int32, (B, B), 0)
    cols = lax.broadcasted_iota(jnp.int32, (B, B), 1)
    eye = rows == cols
    diag_m = jnp.where(eye, dot, 0.0)
    sq_col = jnp.sum(diag_m, axis=1, keepdims=True)
    sq_row = jnp.sum(diag_m, axis=0, keepdims=True)

    same = tcol == trow
    inf = jnp.float32(jnp.inf)
    neg_inf = jnp.float32(-jnp.inf)

    dt = sq_col - 2.0 * dot + sq_row
    dt = jnp.maximum(dt, 0.0)
    distt = jnp.where(dt == 0.0, 0.0, jnp.sqrt(dt))
    dtm_ref[...] = jnp.where(same, inf, distt)

    d = sq_row - 2.0 * dot + sq_col
    d = jnp.maximum(d, 0.0)
    dist = jnp.where(d == 0.0, 0.0, jnp.sqrt(d))

    counts = jnp.sum(same.astype(jnp.float32), axis=1, keepdims=True)
    anchors = (counts < BOUNDARY).astype(jnp.float32)

    curp = jnp.where(same & ~eye, dist, neg_inf)
    prow = []
    for k in range(KPOS):
        pd = jnp.max(curp, axis=1, keepdims=True)
        prow.append(pd)
        if k < KPOS - 1:
            hit = jnp.where(curp == pd, cols, B)
            first = jnp.min(hit, axis=1, keepdims=True)
            curp = jnp.where(cols == first, neg_inf, curp)

    def to_row(v_col):
        return jnp.sum(jnp.where(eye, v_col + jnp.zeros((B, B), jnp.float32),
                                 0.0), axis=0, keepdims=True)

    zero_rows = jnp.zeros((PROWS - KPOS - 1, B), jnp.float32)
    pos_ref[...] = jnp.concatenate(
        [to_row(p) + MARGIN for p in prow] + [to_row(anchors), zero_rows],
        axis=0)


def _col_kernel(dtm_hbm, pos_hbm, ps_hbm, pn_hbm, colblk, pos_v, stage):
    wid = lax.axis_index("s") * NC + lax.axis_index("c")
    base = wid * RW
    pltpu.sync_copy(dtm_hbm.at[pl.ds(base, RW)], colblk)
    pltpu.sync_copy(pos_hbm, pos_v)

    inf = jnp.float32(jnp.inf)
    neg_inf = jnp.float32(-jnp.inf)

    def per_col(j, carry):
        sacc, nacc = carry

        a1 = jnp.full((L,), inf)
        a2 = jnp.full((L,), inf)
        a3 = jnp.full((L,), inf)
        a4 = jnp.full((L,), inf)
        a5 = jnp.full((L,), inf)
        for c in range(CH):
            new = colblk[j, pl.ds(c * L, L)]
            lo = jnp.minimum(a1, new); new = jnp.maximum(a1, new); a1 = lo
            lo = jnp.minimum(a2, new); new = jnp.maximum(a2, new); a2 = lo
            lo = jnp.minimum(a3, new); new = jnp.maximum(a3, new); a3 = lo
            lo = jnp.minimum(a4, new); new = jnp.maximum(a4, new); a4 = lo
            a5 = jnp.minimum(a5, new)
        r, _ = plsc.sort_key_val(a1, a1)
        for ai in (a2, a3, a4, a5):
            s, _ = plsc.sort_key_val(ai, ai)
            r = jnp.minimum(r, lax.rev(s, (0,)))
            r, _ = plsc.sort_key_val(r, r)
        th = _splat_at(r, _iota16() == (KNEG - 1), neg_inf)

        s, n = sacc, nacc
        for c in range(CH):
            dv = colblk[j, pl.ds(c * L, L)]
            anchb = pos_v[KPOS, pl.ds(c * L, L)] > 0.5
            m2 = (dv <= th) & (dv < inf) & anchb
            for k in range(KPOS):
                pk = pos_v[k, pl.ds(c * L, L)]
                tr = jnp.maximum(pk - dv, 0.0)
                s = s + jnp.where(m2, tr, 0.0)
                n = n + jnp.where(m2 & (tr > EPS_POS), 1.0, 0.0)
        return (s, n)

    z = jnp.zeros((L,), jnp.float32)
    sacc, nacc = lax.fori_loop(0, RW, per_col, (z, z))

    stage[...] = sacc
    pltpu.sync_copy(stage, ps_hbm.at[wid])
    stage[...] = nacc
    pltpu.sync_copy(stage, pn_hbm.at[wid])


def _final_kernel(ps_ref, pn_ref, out_ref):
    s = jnp.sum(ps_ref[...])
    n = jnp.sum(pn_ref[...])
    out_ref[...] = (s / (n + EPS_POS)).reshape(1, 1)


_col_phase = functools.partial(
    pl.kernel,
    out_type=(
        jax.ShapeDtypeStruct((NW, L), jnp.float32),
        jax.ShapeDtypeStruct((NW, L), jnp.float32),
    ),
    mesh=_MESH,
    compiler_params=pltpu.CompilerParams(needs_layout_passes=False),
    scratch_types=[
        pltpu.VMEM((RW, B), jnp.float32),
        pltpu.VMEM((PROWS, B), jnp.float32),
        pltpu.VMEM((L,), jnp.float32),
    ],
)(_col_kernel)


def kernel(input, target):
    tcol = target.reshape(B, 1)
    trow = target.reshape(1, B)
    dtm, pos = pl.pallas_call(
        _prep_kernel,
        out_shape=(jax.ShapeDtypeStruct((B, B), jnp.float32),
                   jax.ShapeDtypeStruct((PROWS, B), jnp.float32)),
    )(input, tcol, trow)
    ps, pn = _col_phase(dtm, pos)
    out = pl.pallas_call(
        _final_kernel,
        out_shape=jax.ShapeDtypeStruct((1, 1), jnp.float32),
    )(ps, pn)
    return out.reshape(())


# speedup vs baseline: 2.7984x; 1.0859x over previous
import functools

import jax
import jax.numpy as jnp
from jax import lax
from jax.experimental import pallas as pl
from jax.experimental.pallas import tpu as pltpu
from jax.experimental.pallas import tpu_sc as plsc

B = 512
D = 128
L = 16
NC = 2
NS = 16
NW = NC * NS
RW = B // NW
CH = B // L
PROWS = 8
KPOS = 3
KNEG = 5
MARGIN = 1.0
BOUNDARY = 5.0
EPS_POS = 1e-07

_MESH = plsc.VectorSubcoreMesh(core_axis_name="c", subcore_axis_name="s",
                               num_cores=NC, num_subcores=NS)


def _iota16():
    return lax.broadcasted_iota(jnp.int32, (L,), 0)


def _splat_max(v):
    m = plsc.cummax(v)
    m = lax.rev(m, (0,))
    return plsc.cummax(m)


def _splat_at(v, selj, lowest):
    return _splat_max(jnp.where(selj, v, lowest))


def _prep_kernel(x_ref, tcol_ref, trow_ref, dtm_ref, pos_ref):
    x = x_ref[...]
    tcol = tcol_ref[...]
    trow = trow_ref[...]
    dot = lax.dot_general(x, x, (((1,), (1,)), ((), ())),
                          preferred_element_type=jnp.float32)
    rows = lax.broadcasted_iota(jnp.int32, (B, B), 0)
    cols = lax.broadcasted_iota(jnp.int32, (B, B), 1)
    eye = rows == cols
    diag_m = jnp.where(eye, dot, 0.0)
    sq_col = jnp.sum(diag_m, axis=1, keepdims=True)
    sq_row = jnp.sum(diag_m, axis=0, keepdims=True)

    same = tcol == trow
    inf = jnp.float32(jnp.inf)
    neg_inf = jnp.float32(-jnp.inf)

    dt = sq_col - 2.0 * dot + sq_row
    dt = jnp.maximum(dt, 0.0)
    distt = jnp.where(dt == 0.0, 0.0, jnp.sqrt(dt))
    dtm_ref[...] = jnp.where(same, inf, distt)

    d = sq_row - 2.0 * dot + sq_col
    d = jnp.maximum(d, 0.0)
    dist = jnp.where(d == 0.0, 0.0, jnp.sqrt(d))

    counts = jnp.sum(same.astype(jnp.float32), axis=1, keepdims=True)
    anchors = (counts < BOUNDARY).astype(jnp.float32)

    curp = jnp.where(same & ~eye, dist, neg_inf)
    prow = []
    for k in range(KPOS):
        pd = jnp.max(curp, axis=1, keepdims=True)
        prow.append(pd)
        if k < KPOS - 1:
            hit = jnp.where(curp == pd, cols, B)
            first = jnp.min(hit, axis=1, keepdims=True)
            curp = jnp.where(cols == first, neg_inf, curp)

    def to_row(v_col):
        return jnp.sum(jnp.where(eye, v_col + jnp.zeros((B, B), jnp.float32),
                                 0.0), axis=0, keepdims=True)

    zero_rows = jnp.zeros((PROWS - KPOS - 1, B), jnp.float32)
    pos_ref[...] = jnp.concatenate(
        [to_row(p) + MARGIN for p in prow] + [to_row(anchors), zero_rows],
        axis=0)


def _col_kernel(dtm_hbm, th_hbm, colblk, stage):
    wid = lax.axis_index("s") * NC + lax.axis_index("c")
    base = wid * RW
    pltpu.sync_copy(dtm_hbm.at[pl.ds(base, RW)], colblk)

    inf = jnp.float32(jnp.inf)
    neg_inf = jnp.float32(-jnp.inf)

    def per_col(j, th_all):
        a1 = jnp.full((L,), inf)
        a2 = jnp.full((L,), inf)
        a3 = jnp.full((L,), inf)
        a4 = jnp.full((L,), inf)
        a5 = jnp.full((L,), inf)
        for c in range(CH):
            new = colblk[j, pl.ds(c * L, L)]
            lo = jnp.minimum(a1, new); new = jnp.maximum(a1, new); a1 = lo
            lo = jnp.minimum(a2, new); new = jnp.maximum(a2, new); a2 = lo
            lo = jnp.minimum(a3, new); new = jnp.maximum(a3, new); a3 = lo
            lo = jnp.minimum(a4, new); new = jnp.maximum(a4, new); a4 = lo
            a5 = jnp.minimum(a5, new)
        r, _ = plsc.sort_key_val(a1, a1)
        for ai in (a2, a3, a4, a5):
            s, _ = plsc.sort_key_val(ai, ai)
            r = jnp.minimum(r, lax.rev(s, (0,)))
            r, _ = plsc.sort_key_val(r, r)
        th = _splat_at(r, _iota16() == (KNEG - 1), neg_inf)
        return jnp.where(_iota16() == j, th, th_all)

    th_all = lax.fori_loop(0, RW, per_col, jnp.full((L,), inf))
    stage[...] = th_all
    pltpu.sync_copy(stage, th_hbm.at[wid])


def _final_kernel(dtm_ref, pos_ref, th_ref, out_ref):
    dtm = dtm_ref[...]
    th = th_ref[...]
    anch = pos_ref[pl.ds(KPOS, 1), :]
    inf = jnp.float32(jnp.inf)
    m2 = (dtm <= th) & (dtm < inf) & (anch > 0.5)
    wf = m2.astype(jnp.float32)
    s_total = jnp.float32(0.0)
    n_total = jnp.float32(0.0)
    for k in range(KPOS):
        pk = pos_ref[pl.ds(k, 1), :]
        tr = jnp.maximum(jnp.where(pk > -inf, pk, 0.0) - dtm, 0.0)
        wk = wf * jnp.where(pk > -inf, 1.0, 0.0)
        s_total = s_total + jnp.sum(wk * tr)
        n_total = n_total + jnp.sum(wk * (tr > EPS_POS).astype(jnp.float32))
    out_ref[...] = (s_total / (n_total + EPS_POS)).reshape(1, 1)


_col_phase = functools.partial(
    pl.kernel,
    out_type=jax.ShapeDtypeStruct((NW, L), jnp.float32),
    mesh=_MESH,
    compiler_params=pltpu.CompilerParams(needs_layout_passes=False),
    scratch_types=[
        pltpu.VMEM((RW, B), jnp.float32),
        pltpu.VMEM((L,), jnp.float32),
    ],
)(_col_kernel)


def kernel(input, target):
    tcol = target.reshape(B, 1)
    trow = target.reshape(1, B)
    dtm, pos = pl.pallas_call(
        _prep_kernel,
        out_shape=(jax.ShapeDtypeStruct((B, B), jnp.float32),
                   jax.ShapeDtypeStruct((PROWS, B), jnp.float32)),
    )(input, tcol, trow)
    th = _col_phase(dtm)
    out = pl.pallas_call(
        _final_kernel,
        out_shape=jax.ShapeDtypeStruct((1, 1), jnp.float32),
    )(dtm, pos, th.reshape(B, 1))
    return out.reshape(())
